# baseline (speedup 1.0000x reference)
import functools

import jax
import jax.numpy as jnp
from jax.experimental import pallas as pl
from jax.experimental.pallas import tpu as pltpu

_NPOINT = 1024
_RADIUS = 0.2
_NSAMPLE = 32
_ALPHA = 0.2
_EPS = 1e-5
_N = 4096
_SUB = 8
_LANE = _N // _SUB


def _fps_kernel(xyz_ref, out_ref):
    x = xyz_ref[0]
    y = xyz_ref[1]
    z = xyz_ref[2]
    rows = jax.lax.broadcasted_iota(jnp.int32, (_SUB, _LANE), 0)
    cols = jax.lax.broadcasted_iota(jnp.int32, (_SUB, _LANE), 1)
    lin = rows * _LANE + cols
    s_iota = jax.lax.broadcasted_iota(jnp.int32, (_SUB, _NPOINT), 1)

    def body(t, carry):
        dist, far, out = carry
        fm = lin == far
        cx = jnp.sum(jnp.where(fm, x, 0.0))
        cy = jnp.sum(jnp.where(fm, y, 0.0))
        cz = jnp.sum(jnp.where(fm, z, 0.0))
        dx = x - cx
        dy = y - cy
        dz = z - cz
        d = (dx * dx + dy * dy) + dz * dz
        dist = jnp.minimum(dist, d)
        out = jnp.where(s_iota == t, far, out)
        m = jnp.max(dist)
        far_new = jnp.min(jnp.where(dist == m, lin, jnp.int32(_N)))
        return dist, far_new, out

    dist0 = jnp.full((_SUB, _LANE), 1e10, dtype=jnp.float32)
    out0 = jnp.zeros((_SUB, _NPOINT), dtype=jnp.int32)
    _, _, out = jax.lax.fori_loop(0, _NPOINT, body, (dist0, jnp.int32(0), out0))
    out_ref[...] = out


_NCHUNK = 32
_CLANE = _N // _NCHUNK


def _ball_kernel(xyz_ref, cxyz_ref, out_ref):
    s_blk = cxyz_ref.shape[0]
    x = jnp.reshape(xyz_ref[0], (1, _NCHUNK, _CLANE))
    y = jnp.reshape(xyz_ref[1], (1, _NCHUNK, _CLANE))
    z = jnp.reshape(xyz_ref[2], (1, _NCHUNK, _CLANE))
    cx = jnp.reshape(cxyz_ref[:, 0:1], (s_blk, 1, 1))
    cy = jnp.reshape(cxyz_ref[:, 1:2], (s_blk, 1, 1))
    cz = jnp.reshape(cxyz_ref[:, 2:3], (s_blk, 1, 1))
    bf = jnp.bfloat16
    xb = x.astype(bf).astype(jnp.float32)
    yb = y.astype(bf).astype(jnp.float32)
    zb = z.astype(bf).astype(jnp.float32)
    cxb = cx.astype(bf).astype(jnp.float32)
    cyb = cy.astype(bf).astype(jnp.float32)
    czb = cz.astype(bf).astype(jnp.float32)
    pd = -2.0 * ((cxb * xb + cyb * yb) + czb * zb)
    c2 = (cx * cx + cy * cy) + cz * cz
    x2 = (x * x + y * y) + z * z
    d = (pd + c2) + x2
    mask = jnp.logical_not(d > jnp.float32(_RADIUS * _RADIUS))
    mi = mask.astype(jnp.float32)

    li = jax.lax.broadcasted_iota(jnp.int32, (_CLANE, _CLANE), 0)
    lj = jax.lax.broadcasted_iota(jnp.int32, (_CLANE, _CLANE), 1)
    tri_inc = (li <= lj).astype(jnp.float32)
    mi2 = jnp.reshape(mi, (s_blk * _NCHUNK, _CLANE))
    within = jnp.dot(mi2, tri_inc, preferred_element_type=jnp.float32)
    within = jnp.reshape(within, (s_blk, _NCHUNK, _CLANE))

    sums = jnp.sum(mi, axis=2)
    ci = jax.lax.broadcasted_iota(jnp.int32, (_NCHUNK, _NCHUNK), 0)
    cj = jax.lax.broadcasted_iota(jnp.int32, (_NCHUNK, _NCHUNK), 1)
    tri_exc = (ci < cj).astype(jnp.float32)
    base = jnp.dot(sums, tri_exc, preferred_element_type=jnp.float32)
    base = jnp.reshape(base, (s_blk, _NCHUNK, 1))

    rank = base + within - mi
    lin = (jax.lax.broadcasted_iota(jnp.int32, (s_blk, _NCHUNK, _CLANE), 1) * _CLANE
           + jax.lax.broadcasted_iota(jnp.int32, (s_blk, _NCHUNK, _CLANE), 2))
    big = jnp.int32(_N)
    fv = jnp.min(jnp.min(jnp.where(mask, lin, big), axis=2), axis=1, keepdims=True)
    cnt = jnp.reshape(jnp.sum(sums, axis=1), (s_blk, 1))
    cols = []
    for k in range(_NSAMPLE):
        sel = mask & (rank == jnp.float32(k))
        idx_k = jnp.sum(jnp.sum(jnp.where(sel, lin, 0), axis=2), axis=1,
                        keepdims=True)
        cols.append(jnp.where(cnt > k, idx_k, fv))
    out_ref[...] = jnp.concatenate(cols, axis=1)


def _mlpgat_kernel(g_ref, f_ref, w1_ref, b1_ref, w2_ref, b2_ref, w3_ref,
                   b3_ref, a_ref, out_ref):
    k_n, s_blk, c_in = g_ref.shape
    g = jnp.reshape(g_ref[...], (k_n * s_blk, c_in))
    w1 = w1_ref[...]
    b1 = b1_ref[...]
    w2 = w2_ref[...]
    b2 = b2_ref[...]
    w3 = w3_ref[...]
    b3 = b3_ref[...]

    bf = jnp.bfloat16

    def bdot(u, v):
        return jnp.dot(u.astype(bf), v.astype(bf),
                       preferred_element_type=jnp.float32)

    def mlp(v):
        h = jnp.maximum(bdot(v, w1) + b1, 0.0)
        h = jnp.maximum(bdot(h, w2) + b2, 0.0)
        h = jnp.maximum(bdot(h, w3) + b3, 0.0)
        return h

    h = mlp(g)
    f = mlp(f_ref[...])

    a1 = a_ref[0:3, :]
    a2 = a_ref[3:, :]
    ea = bdot(h, a2)
    fa = bdot(f, a2)
    ep = -(g[:, 0:1] * a1[0:1, :] + g[:, 1:2] * a1[1:2, :] + g[:, 2:3] * a1[2:3, :])

    alpha = jnp.float32(_ALPHA)
    e_list = []
    m = jnp.full((s_blk, 128), -jnp.inf, dtype=jnp.float32)
    for k in range(k_n):
        sl = slice(k * s_blk, (k + 1) * s_blk)
        e_k = ep[sl, :] + fa - ea[sl, :]
        e_k = jnp.where(e_k >= 0.0, e_k, alpha * e_k)
        e_list.append(e_k)
        m = jnp.maximum(m, e_k)
    zsum = jnp.zeros((s_blk, 128), dtype=jnp.float32)
    acc = jnp.zeros((s_blk, 128), dtype=jnp.float32)
    for k in range(k_n):
        sl = slice(k * s_blk, (k + 1) * s_blk)
        p_k = jnp.exp(e_list[k] - m)
        zsum = zsum + p_k
        acc = acc + p_k * h[sl, :]
    out_ref[...] = acc / zsum


def _run_fps(xyz):
    b = xyz.shape[0]
    xyz_r = xyz.reshape(b, 3, _SUB, _LANE)
    out = pl.pallas_call(
        _fps_kernel,
        grid=(b,),
        in_specs=[pl.BlockSpec((None, 3, _SUB, _LANE), lambda i: (i, 0, 0, 0))],
        out_specs=pl.BlockSpec((None, _SUB, _NPOINT), lambda i: (i, 0, 0)),
        out_shape=jax.ShapeDtypeStruct((b, _SUB, _NPOINT), jnp.int32),
        compiler_params=pltpu.CompilerParams(
            dimension_semantics=("parallel",)),
    )(xyz_r)
    return out[:, 0, :]


def _run_ball(xyz, new_xyz, s_blk=128):
    b = xyz.shape[0]
    n_sb = _NPOINT // s_blk
    xyz_c = xyz.reshape(b, 3, _NCHUNK, _CLANE)
    return pl.pallas_call(
        _ball_kernel,
        grid=(b, n_sb),
        in_specs=[
            pl.BlockSpec((None, 3, _NCHUNK, _CLANE), lambda i, j: (i, 0, 0, 0)),
            pl.BlockSpec((None, s_blk, 3), lambda i, j: (i, j, 0)),
        ],
        out_specs=pl.BlockSpec((None, s_blk, _NSAMPLE), lambda i, j: (i, j, 0)),
        out_shape=jax.ShapeDtypeStruct((b, _NPOINT, _NSAMPLE), jnp.int32),
        compiler_params=pltpu.CompilerParams(
            dimension_semantics=("parallel", "parallel")),
    )(xyz_c, new_xyz)


def _run_mlpgat(g, f, w1, b1, w2, b2, w3, b3, a, s_blk=128):
    b, k_n, s, c_in = g.shape
    n_sb = s // s_blk
    c_out = w3.shape[1]
    const = lambda i, j: (0, 0)
    return pl.pallas_call(
        _mlpgat_kernel,
        grid=(b, n_sb),
        in_specs=[
            pl.BlockSpec((None, k_n, s_blk, c_in), lambda i, j: (i, 0, j, 0)),
            pl.BlockSpec((None, s_blk, c_in), lambda i, j: (i, j, 0)),
            pl.BlockSpec(w1.shape, const),
            pl.BlockSpec(b1.shape, const),
            pl.BlockSpec(w2.shape, const),
            pl.BlockSpec(b2.shape, const),
            pl.BlockSpec(w3.shape, const),
            pl.BlockSpec(b3.shape, const),
            pl.BlockSpec(a.shape, const),
        ],
        out_specs=pl.BlockSpec((None, s_blk, c_out), lambda i, j: (i, j, 0)),
        out_shape=jax.ShapeDtypeStruct((b, s, c_out), jnp.float32),
        compiler_params=pltpu.CompilerParams(
            dimension_semantics=("parallel", "parallel")),
    )(g, f, w1, b1, w2, b2, w3, b3, a)


@jax.jit
def kernel(xyz, points, params):
    b = xyz.shape[0]
    xyz_t = xyz.transpose(0, 2, 1)
    pts_t = points.transpose(0, 2, 1)

    fps_idx = _run_fps(xyz)
    new_xyz = jnp.take_along_axis(xyz_t, fps_idx[:, :, None], axis=1)
    idx = _run_ball(xyz, new_xyz)

    idx_t = idx.transpose(0, 2, 1).reshape(b, _NSAMPLE * _NPOINT)
    grouped_xyz = jnp.take_along_axis(xyz_t, idx_t[:, :, None], axis=1)
    grouped_xyz = grouped_xyz.reshape(b, _NSAMPLE, _NPOINT, 3)
    g_norm = grouped_xyz - new_xyz[:, None, :, :]
    grouped_pts = jnp.take_along_axis(pts_t, idx_t[:, :, None], axis=1)
    grouped_pts = grouped_pts.reshape(b, _NSAMPLE, _NPOINT, pts_t.shape[-1])
    g = jnp.concatenate([g_norm, grouped_pts], axis=-1)

    fps_pts = jnp.take_along_axis(pts_t, fps_idx[:, :, None], axis=1)
    f = jnp.concatenate([new_xyz, fps_pts], axis=-1)

    inv = 1.0 / jnp.sqrt(1.0 + _EPS)
    ws, bs = [], []
    for layer in params['convs']:
        scale = inv * layer['gamma']
        ws.append(layer['w'].T * scale[None, :])
        bs.append((layer['b'] * scale + layer['beta'])[None, :])
    a = params['a']

    pooled = _run_mlpgat(g, f, ws[0], bs[0], ws[1], bs[1], ws[2], bs[2], a)
    return new_xyz.transpose(0, 2, 1), pooled.transpose(0, 2, 1)


# speedup vs baseline: 1.2102x; 1.2102x over previous
import functools

import jax
import jax.numpy as jnp
from jax.experimental import pallas as pl
from jax.experimental.pallas import tpu as pltpu

_NPOINT = 1024
_RADIUS = 0.2
_NSAMPLE = 32
_ALPHA = 0.2
_EPS = 1e-5
_N = 4096
_SUB = 8
_LANE = _N // _SUB


def _fps_kernel(xyz_ref, out_ref):
    nb = out_ref.shape[0]
    x = xyz_ref[0]
    y = xyz_ref[1]
    z = xyz_ref[2]
    rows = jax.lax.broadcasted_iota(jnp.int32, (nb, _SUB, _LANE), 1)
    cols = jax.lax.broadcasted_iota(jnp.int32, (nb, _SUB, _LANE), 2)
    lin = rows * _LANE + cols
    s_iota = jax.lax.broadcasted_iota(jnp.int32, (nb, _SUB, _NPOINT), 2)

    def red(v, op):
        return op(op(v, axis=2, keepdims=True), axis=1, keepdims=True)

    def body(t, carry):
        dist, far, out = carry
        fm = lin == far
        cx = red(jnp.where(fm, x, 0.0), jnp.sum)
        cy = red(jnp.where(fm, y, 0.0), jnp.sum)
        cz = red(jnp.where(fm, z, 0.0), jnp.sum)
        dx = x - cx
        dy = y - cy
        dz = z - cz
        d = (dx * dx + dy * dy) + dz * dz
        dist = jnp.minimum(dist, d)
        out = jnp.where(s_iota == t, far, out)
        m = red(dist, jnp.max)
        far_new = red(jnp.where(dist == m, lin, jnp.int32(_N)), jnp.min)
        return dist, far_new, out

    dist0 = jnp.full((nb, _SUB, _LANE), 1e10, dtype=jnp.float32)
    far0 = jnp.zeros((nb, 1, 1), dtype=jnp.int32)
    out0 = jnp.zeros((nb, _SUB, _NPOINT), dtype=jnp.int32)
    _, _, out = jax.lax.fori_loop(0, _NPOINT, body, (dist0, far0, out0))
    out_ref[...] = out


_NCHUNK = 32
_CLANE = _N // _NCHUNK


def _ball_kernel(xyz_ref, cxyz_ref, out_ref):
    s_blk = cxyz_ref.shape[0]
    x = jnp.reshape(xyz_ref[0], (1, _NCHUNK, _CLANE))
    y = jnp.reshape(xyz_ref[1], (1, _NCHUNK, _CLANE))
    z = jnp.reshape(xyz_ref[2], (1, _NCHUNK, _CLANE))
    cx = jnp.reshape(cxyz_ref[:, 0:1], (s_blk, 1, 1))
    cy = jnp.reshape(cxyz_ref[:, 1:2], (s_blk, 1, 1))
    cz = jnp.reshape(cxyz_ref[:, 2:3], (s_blk, 1, 1))
    bf = jnp.bfloat16
    xb = x.astype(bf).astype(jnp.float32)
    yb = y.astype(bf).astype(jnp.float32)
    zb = z.astype(bf).astype(jnp.float32)
    cxb = cx.astype(bf).astype(jnp.float32)
    cyb = cy.astype(bf).astype(jnp.float32)
    czb = cz.astype(bf).astype(jnp.float32)
    pd = -2.0 * ((cxb * xb + cyb * yb) + czb * zb)
    c2 = (cx * cx + cy * cy) + cz * cz
    x2 = (x * x + y * y) + z * z
    d = (pd + c2) + x2
    mask = jnp.logical_not(d > jnp.float32(_RADIUS * _RADIUS))
    mi = mask.astype(jnp.float32)

    li = jax.lax.broadcasted_iota(jnp.int32, (_CLANE, _CLANE), 0)
    lj = jax.lax.broadcasted_iota(jnp.int32, (_CLANE, _CLANE), 1)
    tri_inc = (li <= lj).astype(jnp.float32)
    mi2 = jnp.reshape(mi, (s_blk * _NCHUNK, _CLANE))
    within = jnp.dot(mi2, tri_inc, preferred_element_type=jnp.float32)
    within = jnp.reshape(within, (s_blk, _NCHUNK, _CLANE))

    sums = jnp.sum(mi, axis=2)
    ci = jax.lax.broadcasted_iota(jnp.int32, (_NCHUNK, _NCHUNK), 0)
    cj = jax.lax.broadcasted_iota(jnp.int32, (_NCHUNK, _NCHUNK), 1)
    tri_exc = (ci < cj).astype(jnp.float32)
    base = jnp.dot(sums, tri_exc, preferred_element_type=jnp.float32)
    base = jnp.reshape(base, (s_blk, _NCHUNK, 1))

    rank = base + within - mi
    lin = (jax.lax.broadcasted_iota(jnp.int32, (s_blk, _NCHUNK, _CLANE), 1) * _CLANE
           + jax.lax.broadcasted_iota(jnp.int32, (s_blk, _NCHUNK, _CLANE), 2))
    big = jnp.int32(_N)
    fv = jnp.min(jnp.min(jnp.where(mask, lin, big), axis=2), axis=1, keepdims=True)
    cnt = jnp.reshape(jnp.sum(sums, axis=1), (s_blk, 1))
    cols = []
    for k in range(_NSAMPLE):
        sel = mask & (rank == jnp.float32(k))
        idx_k = jnp.sum(jnp.sum(jnp.where(sel, lin, 0), axis=2), axis=1,
                        keepdims=True)
        cols.append(jnp.where(cnt > k, idx_k, fv))
    out_ref[...] = jnp.concatenate(cols, axis=1)


def _mlpgat_kernel(g_ref, f_ref, w1_ref, b1_ref, w2_ref, b2_ref, w3_ref,
                   b3_ref, a_ref, out_ref):
    k_n, s_blk, c_in = g_ref.shape
    g = jnp.reshape(g_ref[...], (k_n * s_blk, c_in))
    w1 = w1_ref[...]
    b1 = b1_ref[...]
    w2 = w2_ref[...]
    b2 = b2_ref[...]
    w3 = w3_ref[...]
    b3 = b3_ref[...]

    bf = jnp.bfloat16

    def bdot(u, v):
        return jnp.dot(u.astype(bf), v.astype(bf),
                       preferred_element_type=jnp.float32)

    def mlp(v):
        h = jnp.maximum(bdot(v, w1) + b1, 0.0)
        h = jnp.maximum(bdot(h, w2) + b2, 0.0)
        h = jnp.maximum(bdot(h, w3) + b3, 0.0)
        return h

    h = mlp(g)
    f = mlp(f_ref[...])

    a1 = a_ref[0:3, :]
    a2 = a_ref[3:, :]
    ea = bdot(h, a2)
    fa = bdot(f, a2)
    ep = -(g[:, 0:1] * a1[0:1, :] + g[:, 1:2] * a1[1:2, :] + g[:, 2:3] * a1[2:3, :])

    alpha = jnp.float32(_ALPHA)
    e_list = []
    m = jnp.full((s_blk, 128), -jnp.inf, dtype=jnp.float32)
    for k in range(k_n):
        sl = slice(k * s_blk, (k + 1) * s_blk)
        e_k = ep[sl, :] + fa - ea[sl, :]
        e_k = jnp.where(e_k >= 0.0, e_k, alpha * e_k)
        e_list.append(e_k)
        m = jnp.maximum(m, e_k)
    zsum = jnp.zeros((s_blk, 128), dtype=jnp.float32)
    acc = jnp.zeros((s_blk, 128), dtype=jnp.float32)
    for k in range(k_n):
        sl = slice(k * s_blk, (k + 1) * s_blk)
        p_k = jnp.exp(e_list[k] - m)
        zsum = zsum + p_k
        acc = acc + p_k * h[sl, :]
    out_ref[...] = acc / zsum


def _run_fps(xyz):
    b = xyz.shape[0]
    xyz_r = xyz.reshape(b, 3, _SUB, _LANE).transpose(1, 0, 2, 3)
    out = pl.pallas_call(
        _fps_kernel,
        out_shape=jax.ShapeDtypeStruct((b, _SUB, _NPOINT), jnp.int32),
    )(xyz_r)
    return out[:, 0, :]


def _run_ball(xyz, new_xyz, s_blk=128):
    b = xyz.shape[0]
    n_sb = _NPOINT // s_blk
    xyz_c = xyz.reshape(b, 3, _NCHUNK, _CLANE)
    return pl.pallas_call(
        _ball_kernel,
        grid=(b, n_sb),
        in_specs=[
            pl.BlockSpec((None, 3, _NCHUNK, _CLANE), lambda i, j: (i, 0, 0, 0)),
            pl.BlockSpec((None, s_blk, 3), lambda i, j: (i, j, 0)),
        ],
        out_specs=pl.BlockSpec((None, s_blk, _NSAMPLE), lambda i, j: (i, j, 0)),
        out_shape=jax.ShapeDtypeStruct((b, _NPOINT, _NSAMPLE), jnp.int32),
        compiler_params=pltpu.CompilerParams(
            dimension_semantics=("parallel", "parallel")),
    )(xyz_c, new_xyz)


def _run_mlpgat(g, f, w1, b1, w2, b2, w3, b3, a, s_blk=128):
    b, k_n, s, c_in = g.shape
    n_sb = s // s_blk
    c_out = w3.shape[1]
    const = lambda i, j: (0, 0)
    return pl.pallas_call(
        _mlpgat_kernel,
        grid=(b, n_sb),
        in_specs=[
            pl.BlockSpec((None, k_n, s_blk, c_in), lambda i, j: (i, 0, j, 0)),
            pl.BlockSpec((None, s_blk, c_in), lambda i, j: (i, j, 0)),
            pl.BlockSpec(w1.shape, const),
            pl.BlockSpec(b1.shape, const),
            pl.BlockSpec(w2.shape, const),
            pl.BlockSpec(b2.shape, const),
            pl.BlockSpec(w3.shape, const),
            pl.BlockSpec(b3.shape, const),
            pl.BlockSpec(a.shape, const),
        ],
        out_specs=pl.BlockSpec((None, s_blk, c_out), lambda i, j: (i, j, 0)),
        out_shape=jax.ShapeDtypeStruct((b, s, c_out), jnp.float32),
        compiler_params=pltpu.CompilerParams(
            dimension_semantics=("parallel", "parallel")),
    )(g, f, w1, b1, w2, b2, w3, b3, a)


@jax.jit
def kernel(xyz, points, params):
    b = xyz.shape[0]
    xyz_t = xyz.transpose(0, 2, 1)
    pts_t = points.transpose(0, 2, 1)

    fps_idx = _run_fps(xyz)
    new_xyz = jnp.take_along_axis(xyz_t, fps_idx[:, :, None], axis=1)
    idx = _run_ball(xyz, new_xyz)

    idx_t = idx.transpose(0, 2, 1).reshape(b, _NSAMPLE * _NPOINT)
    grouped_xyz = jnp.take_along_axis(xyz_t, idx_t[:, :, None], axis=1)
    grouped_xyz = grouped_xyz.reshape(b, _NSAMPLE, _NPOINT, 3)
    g_norm = grouped_xyz - new_xyz[:, None, :, :]
    grouped_pts = jnp.take_along_axis(pts_t, idx_t[:, :, None], axis=1)
    grouped_pts = grouped_pts.reshape(b, _NSAMPLE, _NPOINT, pts_t.shape[-1])
    g = jnp.concatenate([g_norm, grouped_pts], axis=-1)

    fps_pts = jnp.take_along_axis(pts_t, fps_idx[:, :, None], axis=1)
    f = jnp.concatenate([new_xyz, fps_pts], axis=-1)

    inv = 1.0 / jnp.sqrt(1.0 + _EPS)
    ws, bs = [], []
    for layer in params['convs']:
        scale = inv * layer['gamma']
        ws.append(layer['w'].T * scale[None, :])
        bs.append((layer['b'] * scale + layer['beta'])[None, :])
    a = params['a']

    pooled = _run_mlpgat(g, f, ws[0], bs[0], ws[1], bs[1], ws[2], bs[2], a)
    return new_xyz.transpose(0, 2, 1), pooled.transpose(0, 2, 1)
